# baseline (speedup 1.0000x reference)
import functools

import jax
import jax.numpy as jnp
from jax import lax
from jax.experimental import pallas as pl
from jax.experimental.pallas import tpu as pltpu
from jax.experimental.pallas import tpu_sc as plsc

N = 10000
E = 160000
F0 = 16
H = 8
C = 8
D = H * C
NG = 16

NW = 32
NPT = 313
NPAD = NW * NPT
EPT = 5632
CH = 64
NCH = EPT // CH

_f32 = jnp.float32
_i32 = jnp.int32


def _proj_body(h_ref, wl_ref, wr_ref, wlin_ref, bl_ref, br_ref, blin_ref,
               xlr_ref, lin_ref):
    h = h_ref[...]
    xl = jnp.dot(h, wl_ref[...], preferred_element_type=_f32) + bl_ref[...]
    xr = jnp.dot(h, wr_ref[...], preferred_element_type=_f32) + br_ref[...]
    xlr_ref[...] = jnp.concatenate([xl, xr], axis=1)
    lin_ref[...] = jnp.dot(h, wlin_ref[...], preferred_element_type=_f32) + blin_ref[...]


def _proj(h, wl, wr, wlin, bl, br, blin):
    return pl.pallas_call(
        _proj_body,
        out_shape=[jax.ShapeDtypeStruct((NPAD, 2 * D), _f32),
                   jax.ShapeDtypeStruct((NPAD, D), _f32)],
    )(h, wl, wr, wlin, bl, br, blin)


def _final_body(h_ref, batch_ref, wf_ref, bf_ref, x11_ref, x12_ref):
    oh = (batch_ref[...] == lax.broadcasted_iota(_i32, (NG, NPAD), 0)).astype(_f32)
    ssum = jnp.dot(oh, h_ref[...], preferred_element_type=_f32)
    cnt = jnp.sum(oh, axis=1, keepdims=True)
    x11 = ssum / jnp.maximum(cnt, 1.0)
    x11_ref[...] = x11
    x12_ref[...] = jnp.dot(x11, wf_ref[...], preferred_element_type=_f32) + bf_ref[...]


def _final(h, batch2d, wf, bf):
    return pl.pallas_call(
        _final_body,
        out_shape=[jax.ShapeDtypeStruct((NG, D), _f32),
                   jax.ShapeDtypeStruct((NG, 50000), _f32)],
    )(h, batch2d, wf, bf)


_MESH = plsc.VectorSubcoreMesh(core_axis_name="c", subcore_axis_name="s")

_ACCW = (NPT + 1) * D


@functools.partial(
    pl.kernel,
    out_type=jax.ShapeDtypeStruct((NPAD * D,), _f32),
    mesh=_MESH,
    scratch_types=[
        pltpu.VMEM((EPT,), _i32),
        pltpu.VMEM((EPT,), _i32),
        pltpu.VMEM((EPT,), _f32),
        pltpu.VMEM((EPT,), _f32),
        pltpu.VMEM((CH, 2 * D), _f32),
        pltpu.VMEM((CH, 2 * D), _f32),
        pltpu.VMEM((_ACCW,), _f32),
        pltpu.VMEM((_ACCW,), _f32),
        pltpu.VMEM((_ACCW,), _f32),
        pltpu.VMEM((NPT * D,), _f32),
        pltpu.VMEM((192,), _f32),
        pltpu.SemaphoreType.DMA,
        pltpu.SemaphoreType.DMA,
        pltpu.SemaphoreType.DMA,
    ],
)
def _gat_layer(xlr_hbm, xrf_hbm, linf_hbm, esrc_hbm, edstl_hbm, ea0_hbm,
               ea1_hbm, consts_hbm, out_hbm,
               esrc_l, edstl_l, ea0_l, ea1_l, xj0, xj1, xr_l, acc_l, den_l,
               lin_l, consts_l, sem0, sem1, sem2):
    wid = lax.axis_index("s") * 2 + lax.axis_index("c")
    iota = lax.broadcasted_iota(_i32, (16,), 0)
    zero16 = jnp.zeros((16,), _f32)

    pltpu.sync_copy(esrc_hbm.at[wid], esrc_l)
    pltpu.sync_copy(edstl_hbm.at[wid], edstl_l)
    pltpu.sync_copy(ea0_hbm.at[wid], ea0_l)
    pltpu.sync_copy(ea1_hbm.at[wid], ea1_l)
    pltpu.sync_copy(consts_hbm, consts_l)
    nbase = wid * (NPT * D)
    pltpu.sync_copy(xrf_hbm.at[pl.ds(nbase, NPT * D)], xr_l.at[pl.ds(0, NPT * D)])
    lin_cp = pltpu.make_async_copy(linf_hbm.at[pl.ds(nbase, NPT * D)], lin_l, sem2)
    lin_cp.start()

    att_v = [consts_l[pl.ds(v * 16, 16)] for v in range(4)]
    we0_v = [consts_l[pl.ds(64 + v * 16, 16)] for v in range(4)]
    we1_v = [consts_l[pl.ds(128 + v * 16, 16)] for v in range(4)]
    sh1 = iota ^ 1
    sh2 = iota ^ 2
    sh4 = iota ^ 4

    def _zero(k, _):
        acc_l[pl.ds(k * 16, 16)] = zero16
        den_l[pl.ds(k * 16, 16)] = zero16
        return 0
    lax.fori_loop(0, _ACCW // 16, _zero, 0)

    def _issue(ch, buf, sem):
        pltpu.make_async_copy(
            xlr_hbm.at[esrc_l.at[pl.ds(ch * CH, CH)]], buf, sem).start()

    def _wait(buf, sem):
        pltpu.make_async_copy(xlr_hbm.at[esrc_l.at[pl.ds(0, CH)]], buf, sem).wait()

    _issue(0, xj0, sem0)

    def _process(ch, xjb):
        ebase = ch * CH

        def _group(g, _):
            dv = edstl_l[pl.ds(ebase + g * 16, 16)]
            a0v = ea0_l[pl.ds(ebase + g * 16, 16)]
            a1v = ea1_l[pl.ds(ebase + g * 16, 16)]
            for l in range(16):
                j = dv[l]
                a0 = a0v[l]
                a1 = a1v[l]
                r = g * 16 + l
                nb = j * D
                for v in range(4):
                    xj = xjb[r, pl.ds(v * 16, 16)]
                    xi = xr_l[pl.ds(nb + v * 16, 16)]
                    s = xj + xi + a0 * we0_v[v] + a1 * we1_v[v]
                    z = jnp.maximum(s, 0.2 * s)
                    t = att_v[v] * z
                    t = t + jnp.take(t, sh1)
                    t = t + jnp.take(t, sh2)
                    t = t + jnp.take(t, sh4)
                    ex = jnp.exp(t)
                    plsc.addupdate(den_l.at[pl.ds(nb + v * 16, 16)], ex)
                    plsc.addupdate(acc_l.at[pl.ds(nb + v * 16, 16)], xj * ex)
            return 0
        lax.fori_loop(0, CH // 16, _group, 0)

    def _pair(cc, _):
        ch0 = cc * 2
        _wait(xj0, sem0)
        _issue(ch0 + 1, xj1, sem1)
        _process(ch0, xj0)
        _wait(xj1, sem1)

        @pl.when(ch0 + 2 < NCH)
        def _():
            _issue(ch0 + 2, xj0, sem0)
        _process(ch0 + 1, xj1)
        return 0
    lax.fori_loop(0, NCH // 2, _pair, 0)

    lin_cp.wait()

    def _node(k, _):
        a = acc_l[pl.ds(k * 16, 16)]
        dn = den_l[pl.ds(k * 16, 16)]
        lv = lin_l[pl.ds(k * 16, 16)]
        o = a / (dn + 1e-16) + lv
        o = jnp.where(o > 0, o, jnp.exp(jnp.minimum(o, 0.0)) - 1.0)
        acc_l[pl.ds(k * 16, 16)] = o
        return 0
    lax.fori_loop(0, NPT * D // 16, _node, 0)

    pltpu.sync_copy(acc_l.at[pl.ds(0, NPT * D)], out_hbm.at[pl.ds(nbase, NPT * D)])


def kernel(x, edge_index, edge_attr, batch, params):
    src = edge_index[0]
    dst = edge_index[1]

    order = jnp.argsort(dst)
    dsts = dst[order]
    srcs = src[order]
    eas = edge_attr[order]
    tile = dsts // NPT
    estart = jnp.searchsorted(dsts, jnp.arange(NW, dtype=_i32) * NPT)
    posn = jnp.arange(E, dtype=_i32) - estart[tile].astype(_i32)
    posn = jnp.where(posn < EPT, (posn % NCH) * CH + posn // NCH, NW * EPT)
    flat = tile.astype(_i32) * EPT + posn
    esrc = jnp.zeros((NW * EPT,), _i32).at[flat].set(srcs, mode="drop").reshape(NW, EPT)
    edstl = jnp.full((NW * EPT,), NPT, _i32).at[flat].set(
        dsts - tile * NPT, mode="drop").reshape(NW, EPT)
    ea0 = jnp.zeros((NW * EPT,), _f32).at[flat].set(eas[:, 0], mode="drop").reshape(NW, EPT)
    ea1 = jnp.zeros((NW * EPT,), _f32).at[flat].set(eas[:, 1], mode="drop").reshape(NW, EPT)

    batch2d = jnp.concatenate(
        [batch.astype(_i32), jnp.full((NPAD - N,), NG, _i32)]).reshape(1, NPAD)

    h = jnp.concatenate([x, jnp.zeros((NPAD - N, F0), _f32)], axis=0)
    outs = []
    for p in params['layers']:
        bl = p['bl'].reshape(1, D)
        br = p['br'].reshape(1, D)
        blin = (p['blin'] + p['bias']).reshape(1, D)
        xlr, lin = _proj(h, p['Wl'].T, p['Wr'].T, p['Wlin'].T, bl, br, blin)
        consts = jnp.concatenate(
            [p['att'].reshape(D), p['We'][:, 0], p['We'][:, 1]]).astype(_f32)
        xrf = xlr[:, D:].reshape(-1)
        hf = _gat_layer(xlr, xrf, lin.reshape(-1), esrc, edstl,
                        ea0, ea1, consts)
        h = hf.reshape(NPAD, D)
        outs.append(h[:N])

    x11, x12 = _final(h, batch2d, params['final']['W'].T,
                      params['final']['b'].reshape(1, 50000))
    return tuple(outs) + (x11, x12)


# speedup vs baseline: 1.0146x; 1.0146x over previous
import functools

import jax
import jax.numpy as jnp
from jax import lax
from jax.experimental import pallas as pl
from jax.experimental.pallas import tpu as pltpu
from jax.experimental.pallas import tpu_sc as plsc

N = 10000
E = 160000
F0 = 16
H = 8
C = 8
D = H * C
NG = 16

NW = 32
NPT = 313
NPAD = NW * NPT
EPT = 5632
CH = 64
NCH = EPT // CH

_f32 = jnp.float32
_i32 = jnp.int32


def _proj_body(h_ref, wl_ref, wr_ref, wlin_ref, bl_ref, br_ref, blin_ref,
               xlr_ref, lin_ref):
    h = h_ref[...]
    xl = jnp.dot(h, wl_ref[...], preferred_element_type=_f32) + bl_ref[...]
    xr = jnp.dot(h, wr_ref[...], preferred_element_type=_f32) + br_ref[...]
    xlr_ref[...] = jnp.concatenate([xl, xr], axis=1)
    lin_ref[...] = jnp.dot(h, wlin_ref[...], preferred_element_type=_f32) + blin_ref[...]


def _proj(h, wl, wr, wlin, bl, br, blin):
    return pl.pallas_call(
        _proj_body,
        out_shape=[jax.ShapeDtypeStruct((NPAD, 2 * D), _f32),
                   jax.ShapeDtypeStruct((NPAD, D), _f32)],
    )(h, wl, wr, wlin, bl, br, blin)


def _final_body(h_ref, batch_ref, wf_ref, bf_ref, x11_ref, x12_ref):
    oh = (batch_ref[...] == lax.broadcasted_iota(_i32, (NG, NPAD), 0)).astype(_f32)
    ssum = jnp.dot(oh, h_ref[...], preferred_element_type=_f32)
    cnt = jnp.sum(oh, axis=1, keepdims=True)
    x11 = ssum / jnp.maximum(cnt, 1.0)
    x11_ref[...] = x11
    x12_ref[...] = jnp.dot(x11, wf_ref[...], preferred_element_type=_f32) + bf_ref[...]


def _final(h, batch2d, wf, bf):
    return pl.pallas_call(
        _final_body,
        out_shape=[jax.ShapeDtypeStruct((NG, D), _f32),
                   jax.ShapeDtypeStruct((NG, 50000), _f32)],
    )(h, batch2d, wf, bf)


_MESH = plsc.VectorSubcoreMesh(core_axis_name="c", subcore_axis_name="s")

_ACCW = (NPT + 1) * D
_DENW = (NPT + 1) * 16


@functools.partial(
    pl.kernel,
    out_type=jax.ShapeDtypeStruct((NPAD * D,), _f32),
    mesh=_MESH,
    scratch_types=[
        pltpu.VMEM((EPT,), _i32),
        pltpu.VMEM((EPT,), _i32),
        pltpu.VMEM((EPT,), _f32),
        pltpu.VMEM((EPT,), _f32),
        pltpu.VMEM((CH, 2 * D), _f32),
        pltpu.VMEM((CH, 2 * D), _f32),
        pltpu.VMEM((_ACCW,), _f32),
        pltpu.VMEM((_ACCW,), _f32),
        pltpu.VMEM((_DENW,), _f32),
        pltpu.VMEM((NPT * D,), _f32),
        pltpu.VMEM((192,), _f32),
        pltpu.SemaphoreType.DMA,
        pltpu.SemaphoreType.DMA,
        pltpu.SemaphoreType.DMA,
    ],
)
def _gat_layer(xlr_hbm, xrf_hbm, linf_hbm, esrc_hbm, edstl_hbm, ea0_hbm,
               ea1_hbm, consts_hbm, out_hbm,
               esrc_l, edstl_l, ea0_l, ea1_l, xj0, xj1, xr_l, acc_l, den_l,
               lin_l, consts_l, sem0, sem1, sem2):
    wid = lax.axis_index("s") * 2 + lax.axis_index("c")
    iota = lax.broadcasted_iota(_i32, (16,), 0)
    zero16 = jnp.zeros((16,), _f32)

    pltpu.sync_copy(esrc_hbm.at[wid], esrc_l)
    pltpu.sync_copy(edstl_hbm.at[wid], edstl_l)
    pltpu.sync_copy(ea0_hbm.at[wid], ea0_l)
    pltpu.sync_copy(ea1_hbm.at[wid], ea1_l)
    pltpu.sync_copy(consts_hbm, consts_l)
    nbase = wid * (NPT * D)
    pltpu.sync_copy(xrf_hbm.at[pl.ds(nbase, NPT * D)], xr_l.at[pl.ds(0, NPT * D)])
    lin_cp = pltpu.make_async_copy(linf_hbm.at[pl.ds(nbase, NPT * D)], lin_l, sem2)
    lin_cp.start()

    att_v = [consts_l[pl.ds(v * 16, 16)] for v in range(4)]
    we0_v = [consts_l[pl.ds(64 + v * 16, 16)] for v in range(4)]
    we1_v = [consts_l[pl.ds(128 + v * 16, 16)] for v in range(4)]
    sh8 = iota ^ 8

    def _zero(k, _):
        acc_l[pl.ds(k * 16, 16)] = zero16
        return 0
    lax.fori_loop(0, _ACCW // 16, _zero, 0)

    def _zerod(k, _):
        den_l[pl.ds(k * 16, 16)] = zero16
        return 0
    lax.fori_loop(0, _DENW // 16, _zerod, 0)

    def _issue(ch, buf, sem):
        pltpu.make_async_copy(
            xlr_hbm.at[esrc_l.at[pl.ds(ch * CH, CH)]], buf, sem).start()

    def _wait(buf, sem):
        pltpu.make_async_copy(xlr_hbm.at[esrc_l.at[pl.ds(0, CH)]], buf, sem).wait()

    _issue(0, xj0, sem0)

    def _process(ch, xjb):
        ebase = ch * CH

        def _group(g, _):
            dv = edstl_l[pl.ds(ebase + g * 16, 16)]
            a0v = ea0_l[pl.ds(ebase + g * 16, 16)]
            a1v = ea1_l[pl.ds(ebase + g * 16, 16)]
            for l in range(16):
                j = dv[l]
                a0 = a0v[l]
                a1 = a1v[l]
                r = g * 16 + l
                nb = j * D
                nd = j * 16
                t = None
                xjs = []
                for v in range(4):
                    xj = xjb[r, pl.ds(v * 16, 16)]
                    xi = xr_l[pl.ds(nb + v * 16, 16)]
                    s = xj + xi + a0 * we0_v[v] + a1 * we1_v[v]
                    z = jnp.maximum(s, 0.2 * s)
                    tv = att_v[v] * z
                    t = tv if t is None else t + tv
                    xjs.append(xj)
                t = t + jnp.take(t, sh8)
                ex = jnp.exp(t)
                plsc.addupdate(den_l.at[pl.ds(nd, 16)], ex)
                for v in range(4):
                    plsc.addupdate(acc_l.at[pl.ds(nb + v * 16, 16)],
                                   xjs[v] * ex)
            return 0
        lax.fori_loop(0, CH // 16, _group, 0)

    def _pair(cc, _):
        ch0 = cc * 2
        _wait(xj0, sem0)
        _issue(ch0 + 1, xj1, sem1)
        _process(ch0, xj0)
        _wait(xj1, sem1)

        @pl.when(ch0 + 2 < NCH)
        def _():
            _issue(ch0 + 2, xj0, sem0)
        _process(ch0 + 1, xj1)
        return 0
    lax.fori_loop(0, NCH // 2, _pair, 0)

    lin_cp.wait()

    def _node(k, _):
        a = acc_l[pl.ds(k * 16, 16)]
        dn = den_l[pl.ds((k // 4) * 16, 16)]
        lv = lin_l[pl.ds(k * 16, 16)]
        o = a / (dn + 1e-16) + lv
        o = jnp.where(o > 0, o, jnp.exp(jnp.minimum(o, 0.0)) - 1.0)
        acc_l[pl.ds(k * 16, 16)] = o
        return 0
    lax.fori_loop(0, NPT * D // 16, _node, 0)

    pltpu.sync_copy(acc_l.at[pl.ds(0, NPT * D)], out_hbm.at[pl.ds(nbase, NPT * D)])


def kernel(x, edge_index, edge_attr, batch, params):
    src = edge_index[0]
    dst = edge_index[1]

    _d = jnp.arange(D)
    perm = (_d % C // 2) * 16 + (_d % 2) * 8 + _d // C
    isel = jnp.argsort(perm)

    order = jnp.argsort(dst)
    dsts = dst[order]
    srcs = src[order]
    eas = edge_attr[order]
    tile = dsts // NPT
    estart = jnp.searchsorted(dsts, jnp.arange(NW, dtype=_i32) * NPT)
    posn = jnp.arange(E, dtype=_i32) - estart[tile].astype(_i32)
    posn = jnp.where(posn < EPT, (posn % NCH) * CH + posn // NCH, NW * EPT)
    flat = tile.astype(_i32) * EPT + posn
    esrc = jnp.zeros((NW * EPT,), _i32).at[flat].set(srcs, mode="drop").reshape(NW, EPT)
    edstl = jnp.full((NW * EPT,), NPT, _i32).at[flat].set(
        dsts - tile * NPT, mode="drop").reshape(NW, EPT)
    ea0 = jnp.zeros((NW * EPT,), _f32).at[flat].set(eas[:, 0], mode="drop").reshape(NW, EPT)
    ea1 = jnp.zeros((NW * EPT,), _f32).at[flat].set(eas[:, 1], mode="drop").reshape(NW, EPT)

    batch2d = jnp.concatenate(
        [batch.astype(_i32), jnp.full((NPAD - N,), NG, _i32)]).reshape(1, NPAD)

    h = jnp.concatenate([x, jnp.zeros((NPAD - N, F0), _f32)], axis=0)
    outs = []
    for i, p in enumerate(params['layers']):
        bl = p['bl'][isel].reshape(1, D)
        br = p['br'][isel].reshape(1, D)
        blin = (p['blin'] + p['bias'])[isel].reshape(1, D)
        wl = p['Wl'].T[:, isel]
        wr = p['Wr'].T[:, isel]
        wlin = p['Wlin'].T[:, isel]
        if i > 0:
            wl, wr, wlin = wl[isel], wr[isel], wlin[isel]
        xlr, lin = _proj(h, wl, wr, wlin, bl, br, blin)
        consts = jnp.concatenate(
            [p['att'].reshape(D)[isel], p['We'][:, 0][isel],
             p['We'][:, 1][isel]]).astype(_f32)
        xrf = xlr[:, D:].reshape(-1)
        hf = _gat_layer(xlr, xrf, lin.reshape(-1), esrc, edstl,
                        ea0, ea1, consts)
        h = hf.reshape(NPAD, D)
        outs.append(h[:N][:, perm])

    x11p, x12 = _final(h, batch2d, params['final']['W'].T[isel],
                       params['final']['b'].reshape(1, 50000))
    return tuple(outs) + (x11p[:, perm], x12)
